# Initial kernel scaffold; baseline (speedup 1.0000x reference)
import jax
import jax.numpy as jnp
from jax.experimental import pallas as pl


def kernel(x, edge_index, W_self, b_self, W_neigh, b_neigh):
    raise NotImplementedError("write your pallas kernel here")


# speedup vs baseline: 5.4212x; 5.4212x over previous
import jax
import jax.numpy as jnp
from jax import lax
from jax.experimental import pallas as pl
from jax.experimental.pallas import tpu as pltpu
from jax.experimental.pallas import tpu_sc as plsc

D = 128
DH = 64
N = 10000
E = 320000
NC, NS = 2, 16
K = 128
CHUNKS = 157
EPAD = NS * CHUNKS * K
NPAD = 10112
ZROWS = 632
MMB = 400


def _mm_body(x_ref, wt_ref, b_ref, hs_ref, mg0_ref, mg1_ref):
    y = jnp.dot(x_ref[...], wt_ref[...], preferred_element_type=jnp.float32)
    y = y + b_ref[...]
    hs_ref[...] = y[:, :D]
    mg0_ref[...] = y[:, D:D + DH]
    mg1_ref[...] = y[:, D + DH:]


def _fin_body(hs_ref, p_ref, o_ref):
    p = jnp.concatenate([p_ref[0], p_ref[1]], axis=1)
    o_ref[...] = jnp.maximum(hs_ref[...] + p, 0.0)


def _sc_scatter_body(src_hbm, dst_hbm, msg0_hbm, msg1_hbm, out_hbm,
                     idxs_v, idxd_v, rows_v, zero_v, acc_sh, sem):
    cid = lax.axis_index("c")
    sid = lax.axis_index("s")

    pltpu.sync_copy(src_hbm.at[sid], idxs_v)
    pltpu.sync_copy(dst_hbm.at[sid], idxd_v)

    def _zb(i, carry):
        r = i // 4
        c = (i % 4) * 16
        zero_v[r, pl.ds(c, 16)] = jnp.zeros((16,), jnp.float32)
        return carry
    lax.fori_loop(0, 512, _zb, 0)
    zbase = sid * ZROWS
    for t in range(4):
        pltpu.sync_copy(zero_v, acc_sh.at[pl.ds(zbase + t * 128, 128)])
    pltpu.sync_copy(zero_v.at[pl.ds(0, ZROWS - 512)],
                    acc_sh.at[pl.ds(zbase + 512, ZROWS - 512)])
    plsc.subcore_barrier()

    def _run(msg_hbm):
        def _chunk(j, carry):
            pltpu.async_copy(msg_hbm.at[idxs_v.at[j]], rows_v, sem).wait()
            pltpu.sync_copy(rows_v, acc_sh.at[idxd_v.at[j]], add=True)
            return carry
        lax.fori_loop(0, CHUNKS, _chunk, 0)

    @pl.when(cid == 0)
    def _():
        _run(msg0_hbm)

    @pl.when(cid == 1)
    def _():
        _run(msg1_hbm)

    plsc.subcore_barrier()

    pltpu.sync_copy(acc_sh.at[pl.ds(zbase, ZROWS)],
                    out_hbm.at[cid, pl.ds(zbase, ZROWS)])


@jax.jit
def _sc_scatter(src, dst, msg0, msg1):
    mesh = plsc.VectorSubcoreMesh(core_axis_name="c", subcore_axis_name="s",
                                  num_cores=NC, num_subcores=NS)
    f = pl.kernel(
        _sc_scatter_body,
        out_type=jax.ShapeDtypeStruct((NC, NPAD, DH), jnp.float32),
        mesh=mesh,
        scratch_types=[
            pltpu.VMEM((CHUNKS, K), jnp.int32),
            pltpu.VMEM((CHUNKS, K), jnp.int32),
            pltpu.VMEM((K, DH), jnp.float32),
            pltpu.VMEM((128, DH), jnp.float32),
            pltpu.VMEM_SHARED((NPAD, DH), jnp.float32),
            pltpu.SemaphoreType.DMA,
        ],
        compiler_params=pltpu.CompilerParams(use_tc_tiling_on_sc=False),
    )
    return f(src, dst, msg0, msg1)


@jax.jit
def _mm(x, wt, b):
    return pl.pallas_call(
        _mm_body,
        grid=(N // MMB,),
        in_specs=[
            pl.BlockSpec((MMB, D), lambda i: (i, 0)),
            pl.BlockSpec((D, 2 * D), lambda i: (0, 0)),
            pl.BlockSpec((1, 2 * D), lambda i: (0, 0)),
        ],
        out_specs=[
            pl.BlockSpec((MMB, D), lambda i: (i, 0)),
            pl.BlockSpec((MMB, DH), lambda i: (i, 0)),
            pl.BlockSpec((MMB, DH), lambda i: (i, 0)),
        ],
        out_shape=[
            jax.ShapeDtypeStruct((N, D), jnp.float32),
            jax.ShapeDtypeStruct((N, DH), jnp.float32),
            jax.ShapeDtypeStruct((N, DH), jnp.float32),
        ],
    )(x, wt, b)


@jax.jit
def _finalize(hs, p):
    return pl.pallas_call(
        _fin_body,
        grid=(N // MMB,),
        in_specs=[
            pl.BlockSpec((MMB, D), lambda i: (i, 0)),
            pl.BlockSpec((NC, MMB, DH), lambda i: (0, i, 0)),
        ],
        out_specs=pl.BlockSpec((MMB, D), lambda i: (i, 0)),
        out_shape=jax.ShapeDtypeStruct((N, D), jnp.float32),
    )(hs, p)


def kernel(x, edge_index, W_self, b_self, W_neigh, b_neigh):
    src = edge_index[0].astype(jnp.int32)
    dst = edge_index[1].astype(jnp.int32)
    pad = EPAD - E
    src_p = jnp.concatenate([src, jnp.zeros((pad,), jnp.int32)]).reshape(NS, CHUNKS, K)
    dst_p = jnp.concatenate([dst, jnp.full((pad,), N, jnp.int32)]).reshape(NS, CHUNKS, K)
    wt = jnp.concatenate([W_self.T, W_neigh.T], axis=1)
    b = jnp.concatenate([b_self, b_neigh]).reshape(1, 2 * D)
    hs, msg0, msg1 = _mm(x, wt, b)
    partials = _sc_scatter(src_p, dst_p, msg0, msg1)
    return _finalize(hs, partials)
